# Initial kernel scaffold; baseline (speedup 1.0000x reference)
import jax
import jax.numpy as jnp
from jax.experimental import pallas as pl


def kernel(x, edge_index, W1, b1, W2, b2, W3, b3):
    raise NotImplementedError("write your pallas kernel here")


# speedup vs baseline: 15.8532x; 15.8532x over previous
import functools

import jax
import jax.numpy as jnp
from jax import lax
from jax.experimental import pallas as pl
from jax.experimental.pallas import tpu as pltpu
from jax.experimental.pallas import tpu_sc as plsc

NC = 2
NS = 16
NW = NC * NS
CHUNK = 128


def _sc_mesh():
  return plsc.VectorSubcoreMesh(
      core_axis_name="c", subcore_axis_name="s", num_cores=NC,
      num_subcores=NS)


def _make_deg_kernel(n_chunks, n_pad):
  slab = n_pad // NS

  @functools.partial(
      pl.kernel,
      out_type=jax.ShapeDtypeStruct((NC, n_pad, 16), jnp.float32),
      mesh=_sc_mesh(),
      scratch_types=[
          pltpu.VMEM((n_chunks, CHUNK), jnp.int32),
          pltpu.VMEM((CHUNK, 16), jnp.float32),
          pltpu.VMEM((slab, 16), jnp.float32),
          pltpu.VMEM_SHARED((n_pad, 16), jnp.float32),
      ],
      compiler_params=pltpu.CompilerParams(use_tc_tiling_on_sc=False),
  )
  def deg_kernel(dst3, ones_hbm, zeros_hbm, out, idx_v, ones_v, bounce_v,
                 acc):
    c = lax.axis_index("c")
    s = lax.axis_index("s")
    wid = s * NC + c
    pltpu.sync_copy(dst3.at[wid], idx_v)
    pltpu.sync_copy(ones_hbm, ones_v)
    pltpu.sync_copy(zeros_hbm, bounce_v)
    pltpu.sync_copy(bounce_v, acc.at[pl.ds(s * slab, slab)])
    plsc.subcore_barrier()

    def body(j, carry):
      pltpu.sync_copy(ones_v, acc.at[idx_v.at[j]], add=True)
      return carry

    lax.fori_loop(0, n_chunks, body, 0)
    plsc.subcore_barrier()
    pltpu.sync_copy(acc.at[pl.ds(s * slab, slab)], bounce_v)
    pltpu.sync_copy(bounce_v, out.at[c, pl.ds(s * slab, slab)])

  return deg_kernel


def _make_agg_kernel(n_chunks, n_pad, n, hh):
  slab = n_pad // NS
  q = slab // 4

  @functools.partial(
      pl.kernel,
      out_type=jax.ShapeDtypeStruct((NC, n_pad, hh), jnp.float32),
      mesh=_sc_mesh(),
      scratch_types=[
          pltpu.VMEM((n_chunks, CHUNK), jnp.int32),
          pltpu.VMEM((n_chunks, CHUNK), jnp.int32),
          pltpu.VMEM((CHUNK, hh), jnp.float32),
          pltpu.VMEM((q, hh), jnp.float32),
          pltpu.VMEM_SHARED((n_pad, hh), jnp.float32),
          pltpu.SemaphoreType.DMA,
      ],
      compiler_params=pltpu.CompilerParams(use_tc_tiling_on_sc=False),
  )
  def agg_kernel(src2, dst2, y_hbm, zeros_hbm, out, idxs_v, idxd_v, rows_v,
                 bounce_v, acc, sem):
    c = lax.axis_index("c")
    s = lax.axis_index("s")
    pltpu.sync_copy(src2.at[s], idxs_v)
    pltpu.sync_copy(dst2.at[s], idxd_v)
    for k in range(4):
      pltpu.sync_copy(zeros_hbm, bounce_v)
      pltpu.sync_copy(bounce_v, acc.at[pl.ds(s * slab + k * q, q)])
    plsc.subcore_barrier()

    def body(j, carry):
      pltpu.async_copy(y_hbm.at[c].at[idxs_v.at[j]], rows_v, sem).wait()
      pltpu.sync_copy(rows_v, acc.at[idxd_v.at[j]], add=True)
      return carry

    lax.fori_loop(0, n_chunks, body, 0)
    plsc.subcore_barrier()
    for k in range(4):
      pltpu.sync_copy(acc.at[pl.ds(s * slab + k * q, q)], bounce_v)
      pltpu.sync_copy(bounce_v, out.at[c, pl.ds(s * slab + k * q, q)])

  return agg_kernel


_BLK = 1000


def _dis_block(deg_ref):
  deg = 1.0 + deg_ref[0][:, 0:1] + deg_ref[1][:, 0:1]
  return lax.rsqrt(deg)


def _cat(ref):
  return jnp.concatenate([ref[0], ref[1]], axis=1)


def _split_store(o_ref, v):
  hh = v.shape[1] // 2
  o_ref[0] = v[:, :hh]
  o_ref[1] = v[:, hh:]


def _tc_first_body(deg_ref, x_ref, w_ref, y_ref):
  dis = _dis_block(deg_ref)
  xw = jnp.dot(x_ref[...], w_ref[...], preferred_element_type=jnp.float32)
  _split_store(y_ref, dis * xw)


def _tc_fuse_body(deg_ref, p_ref, y_ref, b_ref, w_ref, o_ref):
  dis = _dis_block(deg_ref)
  h = jax.nn.relu(dis * (_cat(p_ref) + _cat(y_ref)) + b_ref[...])
  _split_store(o_ref, dis * jnp.dot(h, w_ref[...],
                                    preferred_element_type=jnp.float32))


def _tc_final_body(deg_ref, p_ref, y_ref, b_ref, o_ref):
  dis = _dis_block(deg_ref)
  o_ref[...] = jax.nn.relu(dis * (_cat(p_ref) + _cat(y_ref)) + b_ref[...])


def _deg_spec():
  return pl.BlockSpec((NC, _BLK, 16), lambda i: (0, i, 0))


def _half_spec(hh):
  return pl.BlockSpec((NC, _BLK, hh), lambda i: (0, i, 0))


def _tc_first(deg2, x, w, n):
  d_in, h = w.shape
  return pl.pallas_call(
      _tc_first_body,
      grid=(n // _BLK,),
      in_specs=[
          _deg_spec(),
          pl.BlockSpec((_BLK, d_in), lambda i: (i, 0)),
          pl.BlockSpec((d_in, h), lambda i: (0, 0)),
      ],
      out_specs=_half_spec(h // 2),
      out_shape=jax.ShapeDtypeStruct((NC, n, h // 2), jnp.float32),
  )(deg2, x, w)


def _tc_fuse(deg2, p2, y2, b, w, n):
  h_in, h_out = w.shape
  hh = h_in // 2
  return pl.pallas_call(
      _tc_fuse_body,
      grid=(n // _BLK,),
      in_specs=[
          _deg_spec(),
          _half_spec(hh),
          _half_spec(hh),
          pl.BlockSpec((1, h_in), lambda i: (0, 0)),
          pl.BlockSpec((h_in, h_out), lambda i: (0, 0)),
      ],
      out_specs=_half_spec(h_out // 2),
      out_shape=jax.ShapeDtypeStruct((NC, n, h_out // 2), jnp.float32),
  )(deg2, p2, y2, b, w)


def _tc_final(deg2, p2, y2, b, n, h):
  hh = h // 2
  return pl.pallas_call(
      _tc_final_body,
      grid=(n // _BLK,),
      in_specs=[
          _deg_spec(),
          _half_spec(hh),
          _half_spec(hh),
          pl.BlockSpec((1, h), lambda i: (0, 0)),
      ],
      out_specs=pl.BlockSpec((_BLK, h), lambda i: (i, 0)),
      out_shape=jax.ShapeDtypeStruct((n, h), jnp.float32),
  )(deg2, p2, y2, b)


def _pad_edges(idx, parts, fill):
  e = idx.shape[0]
  per = -(-e // (parts * CHUNK)) * CHUNK
  padded = jnp.concatenate(
      [idx, jnp.full((per * parts - e,), fill, jnp.int32)])
  return padded.reshape(parts, per // CHUNK, CHUNK), per // CHUNK


def kernel(x, edge_index, W1, b1, W2, b2, W3, b3):
  n = x.shape[0]
  h1 = W1.shape[1]
  h3 = W3.shape[1]
  n_pad = -(-(n + 1) // (NS * 32)) * (NS * 32)

  src = edge_index[0]
  dst = edge_index[1]
  dst3, nch_d = _pad_edges(dst, NW, n)
  src2, nch_a = _pad_edges(src, NS, 0)
  dst2, _ = _pad_edges(dst, NS, n)

  ones16 = jnp.ones((CHUNK, 16), jnp.float32)
  slab = n_pad // NS
  z16 = jnp.zeros((slab, 16), jnp.float32)
  z32 = jnp.zeros((slab // 4, h1 // 2), jnp.float32)
  z64 = jnp.zeros((slab // 4, h3 // 2), jnp.float32)

  deg2 = _make_deg_kernel(nch_d, n_pad)(dst3, ones16, z16)
  agg_h1 = _make_agg_kernel(nch_a, n_pad, n, h1 // 2)
  agg_h3 = _make_agg_kernel(nch_a, n_pad, n, h3 // 2)

  y1 = _tc_first(deg2, x, W1, n)
  p1 = agg_h1(src2, dst2, y1, z32)
  y2 = _tc_fuse(deg2, p1, y1, b1.reshape(1, -1), W2, n)
  p2 = agg_h1(src2, dst2, y2, z32)
  y3 = _tc_fuse(deg2, p2, y2, b2.reshape(1, -1), W3, n)
  p3 = agg_h3(src2, dst2, y3, z64)
  return _tc_final(deg2, p3, y3, b3.reshape(1, -1), n, h3)


# speedup vs baseline: 16.3222x; 1.0296x over previous
import functools

import jax
import jax.numpy as jnp
from jax import lax
from jax.experimental import pallas as pl
from jax.experimental.pallas import tpu as pltpu
from jax.experimental.pallas import tpu_sc as plsc

NC = 2
NS = 16
NW = NC * NS
CHUNK = 128


def _sc_mesh():
  return plsc.VectorSubcoreMesh(
      core_axis_name="c", subcore_axis_name="s", num_cores=NC,
      num_subcores=NS)


def _make_deg_kernel(n_chunks, n_pad):
  slab = n_pad // NS

  @functools.partial(
      pl.kernel,
      out_type=jax.ShapeDtypeStruct((NC, n_pad, 16), jnp.float32),
      mesh=_sc_mesh(),
      scratch_types=[
          pltpu.VMEM((n_chunks, CHUNK), jnp.int32),
          pltpu.VMEM((CHUNK, 16), jnp.float32),
          pltpu.VMEM((slab, 16), jnp.float32),
          pltpu.VMEM_SHARED((n_pad, 16), jnp.float32),
      ],
      compiler_params=pltpu.CompilerParams(use_tc_tiling_on_sc=False),
  )
  def deg_kernel(dst3, ones_hbm, zeros_hbm, out, idx_v, ones_v, bounce_v,
                 acc):
    c = lax.axis_index("c")
    s = lax.axis_index("s")
    wid = s * NC + c
    pltpu.sync_copy(dst3.at[wid], idx_v)
    pltpu.sync_copy(ones_hbm, ones_v)
    pltpu.sync_copy(zeros_hbm, bounce_v)
    pltpu.sync_copy(bounce_v, acc.at[pl.ds(s * slab, slab)])
    plsc.subcore_barrier()

    def body(j, carry):
      pltpu.sync_copy(ones_v, acc.at[idx_v.at[j]], add=True)
      return carry

    lax.fori_loop(0, n_chunks, body, 0)
    plsc.subcore_barrier()
    pltpu.sync_copy(acc.at[pl.ds(s * slab, slab)], bounce_v)
    pltpu.sync_copy(bounce_v, out.at[c, pl.ds(s * slab, slab)])

  return deg_kernel


def _make_agg_kernel(n_chunks, n_pad, n, hh):
  slab = n_pad // NS
  q = slab // 4

  @functools.partial(
      pl.kernel,
      out_type=jax.ShapeDtypeStruct((NC, n_pad, hh), jnp.float32),
      mesh=_sc_mesh(),
      scratch_types=[
          pltpu.VMEM((n_chunks, CHUNK), jnp.int32),
          pltpu.VMEM((n_chunks, CHUNK), jnp.int32),
          pltpu.VMEM((2, CHUNK, hh), jnp.float32),
          pltpu.VMEM((q, hh), jnp.float32),
          pltpu.VMEM_SHARED((n_pad, hh), jnp.float32),
          pltpu.SemaphoreType.DMA,
          pltpu.SemaphoreType.DMA,
          pltpu.SemaphoreType.DMA,
          pltpu.SemaphoreType.DMA,
      ],
      compiler_params=pltpu.CompilerParams(use_tc_tiling_on_sc=False),
  )
  def agg_kernel(src2, dst2, y_hbm, zeros_hbm, out, idxs_v, idxd_v, rows,
                 bounce_v, acc, gs0, gs1, ss0, ss1):
    c = lax.axis_index("c")
    s = lax.axis_index("s")
    pltpu.sync_copy(src2.at[s], idxs_v)
    pltpu.sync_copy(dst2.at[s], idxd_v)
    for k in range(4):
      pltpu.sync_copy(zeros_hbm, bounce_v)
      pltpu.sync_copy(bounce_v, acc.at[pl.ds(s * slab + k * q, q)])
    plsc.subcore_barrier()

    yc = y_hbm.at[c]

    def gstart(j, b, sem):
      pltpu.async_copy(yc.at[idxs_v.at[j]], rows.at[b], sem)

    def sstart(j, b, sem):
      pltpu.async_copy(rows.at[b], acc.at[idxd_v.at[j]], sem, add=True)

    def drain(b, sem):
      pltpu.make_async_copy(yc.at[pl.ds(0, CHUNK)], rows.at[b], sem).wait()

    nh = n_chunks // 2
    gstart(0, 0, gs0)

    def body(g, carry):
      j0 = 2 * g
      drain(0, gs0)

      @pl.when(g > 0)
      def _():
        drain(1, ss1)

      gstart(j0 + 1, 1, gs1)
      sstart(j0, 0, ss0)
      drain(1, gs1)
      drain(0, ss0)

      @pl.when(g < nh - 1)
      def _():
        gstart(j0 + 2, 0, gs0)

      sstart(j0 + 1, 1, ss1)
      return carry

    lax.fori_loop(0, nh, body, 0)
    drain(1, ss1)
    plsc.subcore_barrier()
    for k in range(4):
      pltpu.sync_copy(acc.at[pl.ds(s * slab + k * q, q)], bounce_v)
      pltpu.sync_copy(bounce_v, out.at[c, pl.ds(s * slab + k * q, q)])

  return agg_kernel


_BLK = 1000


def _dis_block(deg_ref):
  deg = 1.0 + deg_ref[0][:, 0:1] + deg_ref[1][:, 0:1]
  return lax.rsqrt(deg)


def _cat(ref):
  return jnp.concatenate([ref[0], ref[1]], axis=1)


def _split_store(o_ref, v):
  hh = v.shape[1] // 2
  o_ref[0] = v[:, :hh]
  o_ref[1] = v[:, hh:]


def _tc_first_body(deg_ref, x_ref, w_ref, y_ref):
  dis = _dis_block(deg_ref)
  xw = jnp.dot(x_ref[...], w_ref[...], preferred_element_type=jnp.float32)
  _split_store(y_ref, dis * xw)


def _tc_fuse_body(deg_ref, p_ref, y_ref, b_ref, w_ref, o_ref):
  dis = _dis_block(deg_ref)
  h = jax.nn.relu(dis * (_cat(p_ref) + _cat(y_ref)) + b_ref[...])
  _split_store(o_ref, dis * jnp.dot(h, w_ref[...],
                                    preferred_element_type=jnp.float32))


def _tc_final_body(deg_ref, p_ref, y_ref, b_ref, o_ref):
  dis = _dis_block(deg_ref)
  o_ref[...] = jax.nn.relu(dis * (_cat(p_ref) + _cat(y_ref)) + b_ref[...])


def _deg_spec():
  return pl.BlockSpec((NC, _BLK, 16), lambda i: (0, i, 0))


def _half_spec(hh):
  return pl.BlockSpec((NC, _BLK, hh), lambda i: (0, i, 0))


def _tc_first(deg2, x, w, n):
  d_in, h = w.shape
  return pl.pallas_call(
      _tc_first_body,
      grid=(n // _BLK,),
      in_specs=[
          _deg_spec(),
          pl.BlockSpec((_BLK, d_in), lambda i: (i, 0)),
          pl.BlockSpec((d_in, h), lambda i: (0, 0)),
      ],
      out_specs=_half_spec(h // 2),
      out_shape=jax.ShapeDtypeStruct((NC, n, h // 2), jnp.float32),
  )(deg2, x, w)


def _tc_fuse(deg2, p2, y2, b, w, n):
  h_in, h_out = w.shape
  hh = h_in // 2
  return pl.pallas_call(
      _tc_fuse_body,
      grid=(n // _BLK,),
      in_specs=[
          _deg_spec(),
          _half_spec(hh),
          _half_spec(hh),
          pl.BlockSpec((1, h_in), lambda i: (0, 0)),
          pl.BlockSpec((h_in, h_out), lambda i: (0, 0)),
      ],
      out_specs=_half_spec(h_out // 2),
      out_shape=jax.ShapeDtypeStruct((NC, n, h_out // 2), jnp.float32),
  )(deg2, p2, y2, b, w)


def _tc_final(deg2, p2, y2, b, n, h):
  hh = h // 2
  return pl.pallas_call(
      _tc_final_body,
      grid=(n // _BLK,),
      in_specs=[
          _deg_spec(),
          _half_spec(hh),
          _half_spec(hh),
          pl.BlockSpec((1, h), lambda i: (0, 0)),
      ],
      out_specs=pl.BlockSpec((_BLK, h), lambda i: (i, 0)),
      out_shape=jax.ShapeDtypeStruct((n, h), jnp.float32),
  )(deg2, p2, y2, b)


def _pad_edges(idx, parts, fill, chunk_mult=1):
  e = idx.shape[0]
  gran = parts * CHUNK * chunk_mult
  per = (-(-e // gran) * gran) // parts
  padded = jnp.concatenate(
      [idx, jnp.full((per * parts - e,), fill, jnp.int32)])
  return padded.reshape(parts, per // CHUNK, CHUNK), per // CHUNK


def kernel(x, edge_index, W1, b1, W2, b2, W3, b3):
  n = x.shape[0]
  h1 = W1.shape[1]
  h3 = W3.shape[1]
  n_pad = -(-(n + 1) // (NS * 32)) * (NS * 32)

  src = edge_index[0]
  dst = edge_index[1]
  dst3, nch_d = _pad_edges(dst, NW, n)
  src2, nch_a = _pad_edges(src, NS, 0, 2)
  dst2, _ = _pad_edges(dst, NS, n, 2)

  ones16 = jnp.ones((CHUNK, 16), jnp.float32)
  slab = n_pad // NS
  z16 = jnp.zeros((slab, 16), jnp.float32)
  z32 = jnp.zeros((slab // 4, h1 // 2), jnp.float32)
  z64 = jnp.zeros((slab // 4, h3 // 2), jnp.float32)

  deg2 = _make_deg_kernel(nch_d, n_pad)(dst3, ones16, z16)
  agg_h1 = _make_agg_kernel(nch_a, n_pad, n, h1 // 2)
  agg_h3 = _make_agg_kernel(nch_a, n_pad, n, h3 // 2)

  y1 = _tc_first(deg2, x, W1, n)
  p1 = agg_h1(src2, dst2, y1, z32)
  y2 = _tc_fuse(deg2, p1, y1, b1.reshape(1, -1), W2, n)
  p2 = agg_h1(src2, dst2, y2, z32)
  y3 = _tc_fuse(deg2, p2, y2, b2.reshape(1, -1), W3, n)
  p3 = agg_h3(src2, dst2, y3, z64)
  return _tc_final(deg2, p3, y3, b3.reshape(1, -1), n, h3)
